# baseline (speedup 1.0000x reference)
import jax
import jax.numpy as jnp
from jax import lax
from jax.experimental import pallas as pl
from jax.experimental.pallas import tpu as pltpu
from jax.experimental.pallas import tpu_sc as plsc

N_NODES = 10000
N_PAD = 10240
N_EDGES = 160000
E_PAD = 163840
D = 256
DH = 128
NC = 2
NS = 16
TPB = N_PAD // NS
ACC_ROWS = TPB + 1
ACC_LEN = ACC_ROWS * DH
SCH = 1024
NSUP = E_PAD // SCH
LIST_LEN = SCH + 16
DEG_LANES = 16
DEG_LEN = ACC_ROWS * DEG_LANES

_MESH = plsc.VectorSubcoreMesh(core_axis_name="c", subcore_axis_name="s")
_NLP = pltpu.CompilerParams(needs_layout_passes=False)


def _deg_body(dst_hbm, out_hbm, shist, dchunk):
    c = lax.axis_index("c")
    s = lax.axis_index("s")
    lo = s * TPB
    zeros16 = jnp.zeros((16,), jnp.float32)
    ones16 = jnp.ones((16,), jnp.float32)
    iota16 = lax.iota(jnp.int32, 16)

    def z(i, _):
        shist[pl.ds(i * 16, 16)] = zeros16
        return 0
    lax.fori_loop(0, DEG_LEN // 16, z, 0)

    ebase = c * (E_PAD // 2)

    def sup_body(sup, _):
        pltpu.sync_copy(dst_hbm.at[pl.ds(ebase + sup * SCH, SCH)], dchunk)

        def vec(k, _):
            vd = dchunk[pl.ds(k * 16, 16)]
            m = (vd >= lo) & (vd < lo + TPB)
            dstl = jnp.where(m, vd - lo, TPB)
            idx = dstl * DEG_LANES + iota16
            plsc.addupdate_scatter(shist, [idx], ones16, mask=m)
            return 0
        lax.fori_loop(0, SCH // 16, vec, 0)
        return 0
    lax.fori_loop(0, NSUP // NC, sup_body, 0)

    wid = c * NS + s
    pltpu.sync_copy(shist, out_hbm.at[pl.ds(wid * DEG_LEN, DEG_LEN)])


_deg_kernel = pl.kernel(
    _deg_body,
    out_type=jax.ShapeDtypeStruct((NC * NS * DEG_LEN,), jnp.float32),
    mesh=_MESH,
    compiler_params=_NLP,
    scratch_types=[
        pltpu.VMEM((DEG_LEN,), jnp.float32),
        pltpu.VMEM((SCH,), jnp.int32),
    ],
)


GB = 64


def _prop_body(u_hbm, src_hbm, dst_hbm, out_hbm,
               acc, schunk_a, dchunk_a, schunk_b, dchunk_b,
               slist, dlist, rows_a,
               esem_a, esem_b, gsem_a):
    c = lax.axis_index("c")
    s = lax.axis_index("s")
    lo = s * TPB
    cN = c * N_PAD
    zeros16 = jnp.zeros((16,), jnp.float32)
    zeros16i = jnp.zeros((16,), jnp.int32)
    trash16 = jnp.full((16,), TPB, jnp.int32)

    def z(i, _):
        acc[pl.ds(i * 16, 16)] = zeros16
        return 0
    lax.fori_loop(0, ACC_LEN // 16, z, 0)

    def zl(i, _):
        slist[pl.ds(i * 16, 16)] = zeros16i
        return 0
    lax.fori_loop(0, LIST_LEN // 16, zl, 0)

    def fire_edges(sup, schunk, dchunk, esem):
        off = sup * SCH
        pltpu.async_copy(src_hbm.at[pl.ds(off, SCH)], schunk, esem)
        pltpu.async_copy(dst_hbm.at[pl.ds(off, SCH)], dchunk, esem)

    def wait_edges(sup, schunk, dchunk, esem):
        off = sup * SCH
        pltpu.make_async_copy(
            src_hbm.at[pl.ds(off, SCH)], schunk, esem).wait()
        pltpu.make_async_copy(
            dst_hbm.at[pl.ds(off, SCH)], dchunk, esem).wait()

    def gather(g, rows, gsem):
        pltpu.async_copy(
            u_hbm.at[slist.at[pl.ds(g * GB, GB)]], rows, gsem).wait()

    def accumulate(g, rows, n16):
        for sub in range(GB // 16):
            start = g * GB + sub * 16

            @pl.when(start < n16)
            def _():
                dv = dlist[pl.ds(start, 16)]
                for lane in range(16):
                    rowbase = dv[lane] * DH
                    for kk in range(DH // 16):
                        plsc.addupdate(
                            acc.at[pl.ds(rowbase + kk * 16, 16)],
                            rows[sub * 16 + lane, pl.ds(kk * 16, 16)])

    def process(schunk, dchunk):
        def scan_vec(k, n):
            vd = dchunk[pl.ds(k * 16, 16)]
            m = (vd >= lo) & (vd < lo + TPB)
            plsc.store_compressed(dlist.at[pl.ds(n, 16)], vd - lo, mask=m)
            vs = schunk[pl.ds(k * 16, 16)] + cN
            plsc.store_compressed(slist.at[pl.ds(n, 16)], vs, mask=m)
            return n + plsc.all_reduce_population_count(m)[0]
        n = lax.fori_loop(0, SCH // 16, scan_vec, 0)

        full16 = jnp.full((16,), True, jnp.bool_)
        plsc.store_compressed(slist.at[pl.ds(n, 16)], zeros16i, mask=full16)
        plsc.store_compressed(dlist.at[pl.ds(n, 16)], trash16, mask=full16)
        n16 = (n + 15) & ~15
        ng = (n16 + GB - 1) // GB

        def drain(g, _):
            gather(g, rows_a, gsem_a)
            accumulate(g, rows_a, n16)
            return 0
        lax.fori_loop(0, ng, drain, 0)

    def sup_body(sup, _):
        off = sup * SCH
        pltpu.sync_copy(src_hbm.at[pl.ds(off, SCH)], schunk_a)
        pltpu.sync_copy(dst_hbm.at[pl.ds(off, SCH)], dchunk_a)
        process(schunk_a, dchunk_a)
        return 0
    lax.fori_loop(0, NSUP, sup_body, 0)

    obase = (c * N_PAD + s * TPB) * DH
    pltpu.sync_copy(acc.at[pl.ds(0, TPB * DH)],
                    out_hbm.at[pl.ds(obase, TPB * DH)])


_prop_kernel = pl.kernel(
    _prop_body,
    out_type=jax.ShapeDtypeStruct((NC * N_PAD * DH,), jnp.float32),
    mesh=_MESH,
    compiler_params=_NLP,
    scratch_types=[
        pltpu.VMEM((ACC_LEN,), jnp.float32),
        pltpu.VMEM((SCH,), jnp.int32),
        pltpu.VMEM((SCH,), jnp.int32),
        pltpu.VMEM((SCH,), jnp.int32),
        pltpu.VMEM((SCH,), jnp.int32),
        pltpu.VMEM((LIST_LEN,), jnp.int32),
        pltpu.VMEM((LIST_LEN,), jnp.int32),
        pltpu.VMEM((GB, DH), jnp.float32),
        pltpu.SemaphoreType.DMA,
        pltpu.SemaphoreType.DMA,
        pltpu.SemaphoreType.DMA,
    ],
)

_BLK = 256
_GRID = (N_PAD // _BLK,)


def _dis_of(dp_ref):
    deg = jnp.sum(dp_ref[...], axis=1) + 1.0
    return lax.rsqrt(deg)


def _tc1_body(x_ref, w_ref, dp_ref, ul_ref, ur_ref):
    dis = _dis_of(dp_ref)
    h = jnp.dot(x_ref[...], w_ref[...], preferred_element_type=jnp.float32)
    u = h * dis[:, None]
    ul_ref[...] = u[:, :DH]
    ur_ref[...] = u[:, DH:]


def _tc2_body(al_ref, ar_ref, ul_ref, ur_ref, dp_ref, b_ref, w_ref,
              ol_ref, or_ref):
    dis = _dis_of(dp_ref)
    a = jnp.concatenate([al_ref[...], ar_ref[...]], axis=1)
    u = jnp.concatenate([ul_ref[...], ur_ref[...]], axis=1)
    t = (a + u) * dis[:, None] + b_ref[...]
    t = jnp.maximum(t, 0.0)
    u2 = jnp.dot(t, w_ref[...],
                 preferred_element_type=jnp.float32) * dis[:, None]
    ol_ref[...] = u2[:, :DH]
    or_ref[...] = u2[:, DH:]


def _tc3_body(al_ref, ar_ref, ul_ref, ur_ref, dp_ref, b_ref, w_ref, bo_ref,
              o_ref):
    dis = _dis_of(dp_ref)
    a = jnp.concatenate([al_ref[...], ar_ref[...]], axis=1)
    u = jnp.concatenate([ul_ref[...], ur_ref[...]], axis=1)
    t = (a + u) * dis[:, None] + b_ref[...]
    o_ref[...] = jnp.dot(t, w_ref[...],
                         preferred_element_type=jnp.float32) + bo_ref[...]


_row_spec = pl.BlockSpec((_BLK, D), lambda i: (i, 0))
_half_spec = pl.BlockSpec((_BLK, DH), lambda i: (i, 0))
_w_spec = pl.BlockSpec((D, D), lambda i: (0, 0))
_dp_spec = pl.BlockSpec((_BLK, NC * NS), lambda i: (i, 0))
_b_spec = pl.BlockSpec((1, D), lambda i: (0, 0))

_half_out = jax.ShapeDtypeStruct((N_PAD, DH), jnp.float32)

_tc1 = pl.pallas_call(
    _tc1_body, grid=_GRID,
    in_specs=[_row_spec, _w_spec, _dp_spec],
    out_specs=(_half_spec, _half_spec),
    out_shape=(_half_out, _half_out),
)

_tc2 = pl.pallas_call(
    _tc2_body, grid=_GRID,
    in_specs=[_half_spec, _half_spec, _half_spec, _half_spec, _dp_spec,
              _b_spec, _w_spec],
    out_specs=(_half_spec, _half_spec),
    out_shape=(_half_out, _half_out),
)

_tc3 = pl.pallas_call(
    _tc3_body, grid=_GRID,
    in_specs=[_half_spec, _half_spec, _half_spec, _half_spec, _dp_spec,
              _b_spec, _w_spec, _b_spec],
    out_specs=_row_spec,
    out_shape=jax.ShapeDtypeStruct((N_PAD, D), jnp.float32),
)


@jax.jit
def _impl(x, src, dst, W1, b1, W2, b2, Wout, bout):
    srcp = jnp.concatenate(
        [src, jnp.zeros((E_PAD - N_EDGES,), jnp.int32)])
    dstp = jnp.concatenate(
        [dst, jnp.full((E_PAD - N_EDGES,), N_PAD, jnp.int32)])
    x_p = jnp.pad(x, ((0, N_PAD - N_NODES), (0, 0)))

    dparts = _deg_kernel(dstp).reshape(NC * NS, ACC_ROWS, DEG_LANES)
    dparts = dparts[:, :TPB, :]
    dparts = dparts.reshape(NC, NS, TPB, DEG_LANES)
    dp = dparts.transpose(1, 2, 0, 3).reshape(N_PAD, NC * DEG_LANES)

    u1l, u1r = _tc1(x_p, W1, dp)
    u1s = jnp.concatenate([u1l, u1r], axis=0)
    a1 = _prop_kernel(u1s, srcp, dstp).reshape(NC, N_PAD, DH)
    u2l, u2r = _tc2(a1[0], a1[1], u1l, u1r, dp, b1.reshape(1, D), W2)
    u2s = jnp.concatenate([u2l, u2r], axis=0)
    a2 = _prop_kernel(u2s, srcp, dstp).reshape(NC, N_PAD, DH)
    out = _tc3(a2[0], a2[1], u2l, u2r, dp, b2.reshape(1, D), Wout,
               bout.reshape(1, D))
    return out[:N_NODES]


def kernel(x, edge_index, W1, b1, W2, b2, Wout, bout):
    return _impl(x, edge_index[0].astype(jnp.int32),
                 edge_index[1].astype(jnp.int32),
                 W1, b1, W2, b2, Wout, bout)


# speedup vs baseline: 5.8015x; 5.8015x over previous
import jax
import jax.numpy as jnp
from jax import lax
from jax.experimental import pallas as pl
from jax.experimental.pallas import tpu as pltpu
from jax.experimental.pallas import tpu_sc as plsc

N_NODES = 10000
N_PAD = 10240
N_EDGES = 160000
E_PAD = 163840
D = 256
DH = 128
NC = 2
NS = 16
TPB = N_PAD // NS
ACC_ROWS = TPB + 1
ACC_LEN = ACC_ROWS * DH
SCH = 1024
NSUP = E_PAD // SCH
LIST_LEN = SCH + 16
DEG_LANES = 16
DEG_LEN = ACC_ROWS * DEG_LANES

_MESH = plsc.VectorSubcoreMesh(core_axis_name="c", subcore_axis_name="s")
_NLP = pltpu.CompilerParams(needs_layout_passes=False)


def _deg_body(dst_hbm, out_hbm, shist, dchunk):
    c = lax.axis_index("c")
    s = lax.axis_index("s")
    lo = s * TPB
    zeros16 = jnp.zeros((16,), jnp.float32)
    ones16 = jnp.ones((16,), jnp.float32)
    iota16 = lax.iota(jnp.int32, 16)

    def z(i, _):
        shist[pl.ds(i * 16, 16)] = zeros16
        return 0
    lax.fori_loop(0, DEG_LEN // 16, z, 0)

    ebase = c * (E_PAD // 2)

    def sup_body(sup, _):
        pltpu.sync_copy(dst_hbm.at[pl.ds(ebase + sup * SCH, SCH)], dchunk)

        def vec(k, _):
            vd = dchunk[pl.ds(k * 16, 16)]
            m = (vd >= lo) & (vd < lo + TPB)
            dstl = jnp.where(m, vd - lo, TPB)
            idx = dstl * DEG_LANES + iota16
            plsc.addupdate_scatter(shist, [idx], ones16, mask=m)
            return 0
        lax.fori_loop(0, SCH // 16, vec, 0)
        return 0
    lax.fori_loop(0, NSUP // NC, sup_body, 0)

    wid = c * NS + s
    pltpu.sync_copy(shist, out_hbm.at[pl.ds(wid * DEG_LEN, DEG_LEN)])


_deg_kernel = pl.kernel(
    _deg_body,
    out_type=jax.ShapeDtypeStruct((NC * NS * DEG_LEN,), jnp.float32),
    mesh=_MESH,
    compiler_params=_NLP,
    scratch_types=[
        pltpu.VMEM((DEG_LEN,), jnp.float32),
        pltpu.VMEM((SCH,), jnp.int32),
    ],
)


GB = 64


def _prop_body(u_hbm, src_hbm, dst_hbm, out_hbm,
               acc, schunk_a, dchunk_a, schunk_b, dchunk_b,
               slist, dlist, rows_a,
               esem_a, esem_b, gsem_a):
    c = lax.axis_index("c")
    s = lax.axis_index("s")
    lo = s * TPB
    cN = c * N_PAD
    zeros16 = jnp.zeros((16,), jnp.float32)
    zeros16i = jnp.zeros((16,), jnp.int32)
    trash16 = jnp.full((16,), TPB, jnp.int32)

    def z(i, _):
        acc[pl.ds(i * 16, 16)] = zeros16
        return 0
    lax.fori_loop(0, ACC_LEN // 16, z, 0)

    iota16 = lax.iota(jnp.int32, 16)
    spread16 = iota16 * 64 + s * 16

    def zl(i, _):
        slist[pl.ds(i * 16, 16)] = iota16 + i * 16
        return 0
    lax.fori_loop(0, LIST_LEN // 16, zl, 0)

    def fire_edges(sup, schunk, dchunk, esem):
        off = sup * SCH
        pltpu.async_copy(src_hbm.at[pl.ds(off, SCH)], schunk, esem)
        pltpu.async_copy(dst_hbm.at[pl.ds(off, SCH)], dchunk, esem)

    def wait_edges(sup, schunk, dchunk, esem):
        off = sup * SCH
        pltpu.make_async_copy(
            src_hbm.at[pl.ds(off, SCH)], schunk, esem).wait()
        pltpu.make_async_copy(
            dst_hbm.at[pl.ds(off, SCH)], dchunk, esem).wait()

    def gather(g, rows, gsem):
        pltpu.async_copy(
            u_hbm.at[slist.at[pl.ds(g * GB, GB)]], rows, gsem).wait()

    def accumulate(g, rows, n16):
        for sub in range(GB // 16):
            start = g * GB + sub * 16

            @pl.when(start < n16)
            def _():
                dv = dlist[pl.ds(start, 16)]
                for lane in range(16):
                    rowbase = dv[lane] * DH
                    for kk in range(DH // 16):
                        plsc.addupdate(
                            acc.at[pl.ds(rowbase + kk * 16, 16)],
                            rows[sub * 16 + lane, pl.ds(kk * 16, 16)])

    def process(schunk, dchunk):
        def scan_vec(k, n):
            vd = dchunk[pl.ds(k * 16, 16)]
            m = (vd >= lo) & (vd < lo + TPB)
            plsc.store_compressed(dlist.at[pl.ds(n, 16)], vd - lo, mask=m)
            vs = schunk[pl.ds(k * 16, 16)] + cN
            plsc.store_compressed(slist.at[pl.ds(n, 16)], vs, mask=m)
            return n + plsc.all_reduce_population_count(m)[0]
        n = lax.fori_loop(0, SCH // 16, scan_vec, 0)

        full16 = jnp.full((16,), True, jnp.bool_)
        plsc.store_compressed(slist.at[pl.ds(n, 16)], spread16, mask=full16)
        plsc.store_compressed(dlist.at[pl.ds(n, 16)], trash16, mask=full16)
        n16 = (n + 15) & ~15
        ng = (n16 + GB - 1) // GB

        def drain(g, _):
            gather(g, rows_a, gsem_a)
            accumulate(g, rows_a, n16)
            return 0
        lax.fori_loop(0, ng, drain, 0)

    def sup_body(sup, _):
        off = sup * SCH
        pltpu.sync_copy(src_hbm.at[pl.ds(off, SCH)], schunk_a)
        pltpu.sync_copy(dst_hbm.at[pl.ds(off, SCH)], dchunk_a)
        process(schunk_a, dchunk_a)
        return 0
    lax.fori_loop(0, NSUP, sup_body, 0)

    obase = (c * N_PAD + s * TPB) * DH
    pltpu.sync_copy(acc.at[pl.ds(0, TPB * DH)],
                    out_hbm.at[pl.ds(obase, TPB * DH)])


_prop_kernel = pl.kernel(
    _prop_body,
    out_type=jax.ShapeDtypeStruct((NC * N_PAD * DH,), jnp.float32),
    mesh=_MESH,
    compiler_params=_NLP,
    scratch_types=[
        pltpu.VMEM((ACC_LEN,), jnp.float32),
        pltpu.VMEM((SCH,), jnp.int32),
        pltpu.VMEM((SCH,), jnp.int32),
        pltpu.VMEM((SCH,), jnp.int32),
        pltpu.VMEM((SCH,), jnp.int32),
        pltpu.VMEM((LIST_LEN,), jnp.int32),
        pltpu.VMEM((LIST_LEN,), jnp.int32),
        pltpu.VMEM((GB, DH), jnp.float32),
        pltpu.SemaphoreType.DMA,
        pltpu.SemaphoreType.DMA,
        pltpu.SemaphoreType.DMA,
    ],
)

_BLK = 256
_GRID = (N_PAD // _BLK,)


def _dis_of(dp_ref):
    deg = jnp.sum(dp_ref[...], axis=1) + 1.0
    return lax.rsqrt(deg)


def _tc1_body(x_ref, w_ref, dp_ref, ul_ref, ur_ref):
    dis = _dis_of(dp_ref)
    h = jnp.dot(x_ref[...], w_ref[...], preferred_element_type=jnp.float32)
    u = h * dis[:, None]
    ul_ref[...] = u[:, :DH]
    ur_ref[...] = u[:, DH:]


def _tc2_body(al_ref, ar_ref, ul_ref, ur_ref, dp_ref, b_ref, w_ref,
              ol_ref, or_ref):
    dis = _dis_of(dp_ref)
    a = jnp.concatenate([al_ref[...], ar_ref[...]], axis=1)
    u = jnp.concatenate([ul_ref[...], ur_ref[...]], axis=1)
    t = (a + u) * dis[:, None] + b_ref[...]
    t = jnp.maximum(t, 0.0)
    u2 = jnp.dot(t, w_ref[...],
                 preferred_element_type=jnp.float32) * dis[:, None]
    ol_ref[...] = u2[:, :DH]
    or_ref[...] = u2[:, DH:]


def _tc3_body(al_ref, ar_ref, ul_ref, ur_ref, dp_ref, b_ref, w_ref, bo_ref,
              o_ref):
    dis = _dis_of(dp_ref)
    a = jnp.concatenate([al_ref[...], ar_ref[...]], axis=1)
    u = jnp.concatenate([ul_ref[...], ur_ref[...]], axis=1)
    t = (a + u) * dis[:, None] + b_ref[...]
    o_ref[...] = jnp.dot(t, w_ref[...],
                         preferred_element_type=jnp.float32) + bo_ref[...]


_row_spec = pl.BlockSpec((_BLK, D), lambda i: (i, 0))
_half_spec = pl.BlockSpec((_BLK, DH), lambda i: (i, 0))
_w_spec = pl.BlockSpec((D, D), lambda i: (0, 0))
_dp_spec = pl.BlockSpec((_BLK, NC * NS), lambda i: (i, 0))
_b_spec = pl.BlockSpec((1, D), lambda i: (0, 0))

_half_out = jax.ShapeDtypeStruct((N_PAD, DH), jnp.float32)

_tc1 = pl.pallas_call(
    _tc1_body, grid=_GRID,
    in_specs=[_row_spec, _w_spec, _dp_spec],
    out_specs=(_half_spec, _half_spec),
    out_shape=(_half_out, _half_out),
)

_tc2 = pl.pallas_call(
    _tc2_body, grid=_GRID,
    in_specs=[_half_spec, _half_spec, _half_spec, _half_spec, _dp_spec,
              _b_spec, _w_spec],
    out_specs=(_half_spec, _half_spec),
    out_shape=(_half_out, _half_out),
)

_tc3 = pl.pallas_call(
    _tc3_body, grid=_GRID,
    in_specs=[_half_spec, _half_spec, _half_spec, _half_spec, _dp_spec,
              _b_spec, _w_spec, _b_spec],
    out_specs=_row_spec,
    out_shape=jax.ShapeDtypeStruct((N_PAD, D), jnp.float32),
)


@jax.jit
def _impl(x, src, dst, W1, b1, W2, b2, Wout, bout):
    srcp = jnp.concatenate(
        [src, jnp.zeros((E_PAD - N_EDGES,), jnp.int32)])
    dstp = jnp.concatenate(
        [dst, jnp.full((E_PAD - N_EDGES,), N_PAD, jnp.int32)])
    x_p = jnp.pad(x, ((0, N_PAD - N_NODES), (0, 0)))

    dparts = _deg_kernel(dstp).reshape(NC * NS, ACC_ROWS, DEG_LANES)
    dparts = dparts[:, :TPB, :]
    dparts = dparts.reshape(NC, NS, TPB, DEG_LANES)
    dp = dparts.transpose(1, 2, 0, 3).reshape(N_PAD, NC * DEG_LANES)

    u1l, u1r = _tc1(x_p, W1, dp)
    u1s = jnp.concatenate([u1l, u1r], axis=0)
    a1 = _prop_kernel(u1s, srcp, dstp).reshape(NC, N_PAD, DH)
    u2l, u2r = _tc2(a1[0], a1[1], u1l, u1r, dp, b1.reshape(1, D), W2)
    u2s = jnp.concatenate([u2l, u2r], axis=0)
    a2 = _prop_kernel(u2s, srcp, dstp).reshape(NC, N_PAD, DH)
    out = _tc3(a2[0], a2[1], u2l, u2r, dp, b2.reshape(1, D), Wout,
               bout.reshape(1, D))
    return out[:N_NODES]


def kernel(x, edge_index, W1, b1, W2, b2, Wout, bout):
    return _impl(x, edge_index[0].astype(jnp.int32),
                 edge_index[1].astype(jnp.int32),
                 W1, b1, W2, b2, Wout, bout)
